# Initial kernel scaffold; baseline (speedup 1.0000x reference)
import jax
import jax.numpy as jnp
from jax.experimental import pallas as pl


def kernel(vertices, edge_index, current_vertex, W1c, b1c, W2c, b2c, W3c, b3c, Wm1, bm1, Wm2, bm2):
    raise NotImplementedError("write your pallas kernel here")


# speedup vs baseline: 7.6671x; 7.6671x over previous
import functools

import jax
import jax.numpy as jnp
from jax import lax
from jax.experimental import pallas as pl
from jax.experimental.pallas import tpu as pltpu
from jax.experimental.pallas import tpu_sc as plsc

N = 10000
E = 320000
D = 128
H = 256
NPAD = 10240
NW = 32
ER = E // 128
RPW = -(-ER // NW)
ERP = RPW * NW
ZS = NPAD // 16
CH = 1024
NCH = NPAD // CH


def _mask_body(src_hbm, dst_hbm, cv_hbm, out_hbm, src_v, dbuf, cbuf, zbuf, cv_v, shared):
    c = lax.axis_index("c")
    s = lax.axis_index("s")
    wid = s * 2 + c
    pltpu.sync_copy(src_hbm.at[wid], src_v)
    pltpu.sync_copy(dst_hbm.at[wid], dbuf)
    pltpu.sync_copy(cv_hbm, cv_v)
    cvv = cv_v[...]
    zeros = jnp.zeros((16,), jnp.float32)
    ones = jnp.ones((16,), jnp.float32)

    def zero_body(i, carry):
        zbuf[pl.ds(i * 16, 16)] = zeros
        return carry

    lax.fori_loop(0, ZS // 16, zero_body, 0)
    pltpu.sync_copy(zbuf, shared.at[pl.ds(s * ZS, ZS)])
    plsc.subcore_barrier()

    def row_body(j, carry):
        nhit = jnp.zeros((16,), jnp.int32)
        for k in range(8):
            sv = src_v[j, pl.ds(k * 16, 16)]
            hit = sv == cvv
            contrib = jnp.where(hit, ones, zeros)
            cbuf[j, pl.ds(k * 16, 16)] = contrib
            nhit = nhit + plsc.all_reduce_population_count(hit)
        any_hit = nhit[0]

        @pl.when(any_hit > 0)
        def _():
            pltpu.sync_copy(cbuf.at[j], shared.at[dbuf.at[j]], add=True)

        return carry

    lax.fori_loop(0, RPW, row_body, 0)
    plsc.subcore_barrier()

    @pl.when(s == 0)
    def _():
        pltpu.sync_copy(shared, out_hbm.at[c])


@functools.cache
def _mask_kernel():
    return pl.kernel(
        _mask_body,
        mesh=plsc.VectorSubcoreMesh(core_axis_name="c", subcore_axis_name="s"),
        compiler_params=pltpu.CompilerParams(needs_layout_passes=False),
        out_type=jax.ShapeDtypeStruct((2, NPAD), jnp.float32),
        scratch_types=[
            pltpu.VMEM((RPW, 128), jnp.int32),
            pltpu.VMEM((RPW, 128), jnp.int32),
            pltpu.VMEM((RPW, 128), jnp.float32),
            pltpu.VMEM((ZS,), jnp.float32),
            pltpu.VMEM((16,), jnp.int32),
            pltpu.VMEM_SHARED((NPAD,), jnp.float32),
        ],
    )


def _tc_body(v_ref, w1_ref, b1_ref, w2t_ref, b2_ref, part_ref, out_ref, scores):
    def chunk_body(c, carry):
        vch = v_ref[pl.ds(c * CH, CH), :]
        h = lax.dot_general(
            w1_ref[...], vch, (((0,), (1,)), ((), ())),
            preferred_element_type=jnp.float32,
        )
        h = h + b1_ref[...]
        h = jnp.where(h > 0, h, 0.1 * h)
        s = lax.dot_general(
            w2t_ref[...], h, (((1,), (0,)), ((), ())),
            preferred_element_type=jnp.float32,
        )
        scores[0, pl.ds(c * CH, CH)] = s[0, :] + b2_ref[0, 0]
        return carry

    lax.fori_loop(0, NCH, chunk_body, 0)

    nbr = jnp.sum(part_ref[...], axis=0, keepdims=True)
    idx = lax.broadcasted_iota(jnp.int32, (1, NPAD), 1)
    sc = scores[...]
    logits = jnp.where(idx < N, jnp.where(nbr > 0, sc, -1e9), -jnp.inf)
    m = jnp.max(logits)
    e = jnp.exp(logits - m)
    out_ref[...] = e / jnp.sum(e)


def _tc_call(v_pad, w1, b1_col, w2t, b2, partials):
    return pl.pallas_call(
        _tc_body,
        out_shape=jax.ShapeDtypeStruct((1, NPAD), jnp.float32),
        scratch_shapes=[pltpu.VMEM((1, NPAD), jnp.float32)],
    )(v_pad, w1, b1_col, w2t, b2, partials)


def kernel(vertices, edge_index, current_vertex,
           W1c, b1c, W2c, b2c, W3c, b3c, Wm1, bm1, Wm2, bm2):
    ei_pad = jnp.pad(edge_index, ((0, 0), (0, (ERP - ER) * 128)),
                     constant_values=-1)
    src = ei_pad[0].reshape(NW, RPW, 128)
    dst = ei_pad[1].reshape(NW, RPW, 128)
    cv_vec = jnp.full((16,), current_vertex, dtype=jnp.int32)
    partials = _mask_kernel()(src, dst, cv_vec)

    v = vertices.astype(jnp.float32)
    v_pad = jnp.pad(v, ((0, NPAD - N), (0, 0)))
    b1_col = bm1.reshape(H, 1)
    w2t = Wm2.reshape(1, H)
    b2 = bm2.reshape(1, 1)
    probs = _tc_call(v_pad, Wm1, b1_col, w2t, b2, partials)
    return probs[0, :N]
